# baseline (speedup 1.0000x reference)
import functools

import jax
import jax.numpy as jnp
from jax import lax
from jax.experimental import pallas as pl
from jax.experimental.pallas import tpu as pltpu
from jax.experimental.pallas import tpu_sc as plsc

N = 10000
E = 320000
NPAD = 10112
K = 128
CHUNKS = E // K
DEGW = 16
NBUF = 8
PD = 6


def _make_agg(d):
    info = plsc.get_sparse_core_info()
    nc, ns = info.num_cores, info.num_subcores
    nw = nc * ns
    cw = CHUNKS // nw
    nx = CHUNKS - cw * nw
    rpt = NPAD // ns

    mesh = plsc.VectorSubcoreMesh(core_axis_name="c", subcore_axis_name="s")

    @functools.partial(
        pl.kernel,
        mesh=mesh,
        out_type=jax.ShapeDtypeStruct((nc, NPAD, d), jnp.float32),
        compiler_params=pltpu.CompilerParams(use_tc_tiling_on_sc=False),
        scratch_types=[
            pltpu.VMEM((cw, K), jnp.int32),
            pltpu.VMEM((cw, K), jnp.int32),
            pltpu.VMEM((1, K), jnp.int32),
            pltpu.VMEM((1, K), jnp.int32),
            pltpu.VMEM((NBUF, K, d), jnp.float32),
            pltpu.VMEM_SHARED((NPAD, d), jnp.float32),
            pltpu.SemaphoreType.DMA((NBUF,)),
        ],
    )
    def agg(src_hbm, dst_hbm, zs_hbm, zeros_hbm, out_hbm, sidx, didx, sidx_x,
            didx_x, rows, acc, gsem):
        c = lax.axis_index("c")
        s = lax.axis_index("s")
        wid = c * ns + s
        pltpu.sync_copy(zeros_hbm.at[pl.ds(s * rpt, rpt)],
                        acc.at[pl.ds(s * rpt, rpt)])
        pltpu.sync_copy(src_hbm.at[pl.ds(wid * cw, cw)], sidx)
        pltpu.sync_copy(dst_hbm.at[pl.ds(wid * cw, cw)], didx)

        @pl.when(wid < nx)
        def _():
            pltpu.sync_copy(src_hbm.at[pl.ds(cw * nw + wid, 1)], sidx_x)
            pltpu.sync_copy(dst_hbm.at[pl.ds(cw * nw + wid, 1)], didx_x)

        plsc.subcore_barrier()

        for b in range(PD):
            pltpu.async_copy(zs_hbm.at[sidx.at[b]], rows.at[b], gsem.at[b])

        def body(j, carry):
            jb = j + PD
            b2 = lax.rem(jb, NBUF)

            @pl.when(jb < cw)
            def _():
                pltpu.async_copy(zs_hbm.at[sidx.at[jb]], rows.at[b2],
                                 gsem.at[b2])

            b = lax.rem(j, NBUF)
            pltpu.make_async_copy(zs_hbm.at[sidx.at[j]], rows.at[b],
                                  gsem.at[b]).wait()
            pltpu.sync_copy(rows.at[b], acc.at[didx.at[j]], add=True)
            return carry

        lax.fori_loop(0, cw, body, 0)

        @pl.when(wid < nx)
        def _():
            pltpu.sync_copy(zs_hbm.at[sidx_x.at[0]], rows.at[0])
            pltpu.sync_copy(rows.at[0], acc.at[didx_x.at[0]], add=True)

        plsc.subcore_barrier()
        pltpu.sync_copy(acc.at[pl.ds(s * rpt, rpt)],
                        out_hbm.at[c, pl.ds(s * rpt, rpt)])

    return agg


def _make_deg():
    info = plsc.get_sparse_core_info()
    nc, ns = info.num_cores, info.num_subcores
    nw = nc * ns
    cw = CHUNKS // nw
    nx = CHUNKS - cw * nw
    rpt = NPAD // ns

    mesh = plsc.VectorSubcoreMesh(core_axis_name="c", subcore_axis_name="s")

    @functools.partial(
        pl.kernel,
        mesh=mesh,
        out_type=jax.ShapeDtypeStruct((nc, NPAD, DEGW), jnp.float32),
        compiler_params=pltpu.CompilerParams(use_tc_tiling_on_sc=False),
        scratch_types=[
            pltpu.VMEM((cw, K), jnp.int32),
            pltpu.VMEM((1, K), jnp.int32),
            pltpu.VMEM((K, DEGW), jnp.float32),
            pltpu.VMEM_SHARED((NPAD, DEGW), jnp.float32),
        ],
    )
    def deg(dst_hbm, ones_hbm, zeros_hbm, out_hbm, didx, didx_x, ones, acc):
        c = lax.axis_index("c")
        s = lax.axis_index("s")
        wid = c * ns + s
        pltpu.sync_copy(zeros_hbm.at[pl.ds(s * rpt, rpt)],
                        acc.at[pl.ds(s * rpt, rpt)])
        pltpu.sync_copy(dst_hbm.at[pl.ds(wid * cw, cw)], didx)
        pltpu.sync_copy(ones_hbm, ones)

        @pl.when(wid < nx)
        def _():
            pltpu.sync_copy(dst_hbm.at[pl.ds(cw * nw + wid, 1)], didx_x)

        plsc.subcore_barrier()

        def body(j, carry):
            pltpu.sync_copy(ones, acc.at[didx.at[j]], add=True)
            return carry

        lax.fori_loop(0, cw, body, 0)

        @pl.when(wid < nx)
        def _():
            pltpu.sync_copy(ones, acc.at[didx_x.at[0]], add=True)

        plsc.subcore_barrier()
        pltpu.sync_copy(acc.at[pl.ds(s * rpt, rpt)],
                        out_hbm.at[c, pl.ds(s * rpt, rpt)])

    return deg


def _tc1a_body(x_ref, w_ref, z_ref):
    z_ref[...] = jnp.dot(x_ref[...], w_ref[...],
                         preferred_element_type=jnp.float32)


def _tc1b_body(degp_ref, z_ref, zs_ref, dinv_ref):
    deg = degp_ref[0] + degp_ref[1] + 1.0
    dinv = lax.rsqrt(deg)
    dinv_ref[...] = dinv
    zs_ref[0:N, :] = dinv[0:N] * z_ref[...]
    zs_ref[N:NPAD, :] = jnp.zeros((NPAD - N, 64), jnp.float32)


def _tc2_body(acc_ref, zs_ref, dinv_ref, b_ref, w_ref, out_ref):
    dinv = dinv_ref[...]
    agg = acc_ref[0] + acc_ref[1] + zs_ref[...]
    h = jnp.maximum(dinv * agg + b_ref[...], 0.0)
    z2 = jnp.dot(h[0:N], w_ref[...], preferred_element_type=jnp.float32)
    out_ref[0:N, :] = dinv[0:N] * z2
    out_ref[N:NPAD, :] = jnp.zeros((NPAD - N, 32), jnp.float32)


def _tc3_body(acc_ref, zs_ref, dinv_ref, b_ref, wc_ref, bc_ref, out_ref):
    dinv = dinv_ref[0:N]
    agg = acc_ref[0, 0:N] + acc_ref[1, 0:N] + zs_ref[0:N]
    h = jnp.maximum(dinv * agg + b_ref[...], 0.0)
    o = jnp.dot(h, wc_ref[...], preferred_element_type=jnp.float32) + bc_ref[...]
    out_ref[...] = jax.nn.sigmoid(o)


def kernel(x, edge_index, W1, b1, W2, b2, Wc, bc):
    src_p = edge_index[0].reshape(CHUNKS, K)
    dst_p = edge_index[1].reshape(CHUNKS, K)

    zeros64 = jnp.zeros((NPAD, 64), jnp.float32)
    zeros32 = jnp.zeros((NPAD, 32), jnp.float32)
    zerosd = jnp.zeros((NPAD, DEGW), jnp.float32)
    onesd = jnp.ones((K, DEGW), jnp.float32)

    degp = _make_deg()(dst_p, onesd, zerosd)[:, :, 0:1]

    tc1a = pl.pallas_call(
        _tc1a_body,
        out_shape=jax.ShapeDtypeStruct((N, 64), jnp.float32),
    )
    z1 = tc1a(x, W1)

    tc1b = pl.pallas_call(
        _tc1b_body,
        out_shape=(jax.ShapeDtypeStruct((NPAD, 64), jnp.float32),
                   jax.ShapeDtypeStruct((NPAD, 1), jnp.float32)),
    )
    zs1, dinv = tc1b(degp, z1)

    acc1 = _make_agg(64)(src_p, dst_p, zs1, zeros64)

    tc2 = pl.pallas_call(
        _tc2_body,
        out_shape=jax.ShapeDtypeStruct((NPAD, 32), jnp.float32),
    )
    zs2 = tc2(acc1, zs1, dinv, b1.reshape(1, 64), W2)

    acc2 = _make_agg(32)(src_p, dst_p, zs2, zeros32)

    tc3 = pl.pallas_call(
        _tc3_body,
        out_shape=jax.ShapeDtypeStruct((N, 1), jnp.float32),
    )
    return tc3(acc2, zs2, dinv, b2.reshape(1, 32), Wc, bc.reshape(1, 1))


# speedup vs baseline: 1.0127x; 1.0127x over previous
import functools

import jax
import jax.numpy as jnp
from jax import lax
from jax.experimental import pallas as pl
from jax.experimental.pallas import tpu as pltpu
from jax.experimental.pallas import tpu_sc as plsc

N = 10000
E = 320000
NPAD = 10112
K = 128
CHUNKS = E // K
DEGW = 8
NBUF = 8
PD = 4


def _make_agg(d):
    info = plsc.get_sparse_core_info()
    nc, ns = info.num_cores, info.num_subcores
    nw = nc * ns
    cw = CHUNKS // nw
    nx = CHUNKS - cw * nw
    rpt = NPAD // ns

    mesh = plsc.VectorSubcoreMesh(core_axis_name="c", subcore_axis_name="s")

    @functools.partial(
        pl.kernel,
        mesh=mesh,
        out_type=jax.ShapeDtypeStruct((nc, NPAD, d), jnp.float32),
        compiler_params=pltpu.CompilerParams(use_tc_tiling_on_sc=False),
        scratch_types=[
            pltpu.VMEM((cw, K), jnp.int32),
            pltpu.VMEM((cw, K), jnp.int32),
            pltpu.VMEM((1, K), jnp.int32),
            pltpu.VMEM((1, K), jnp.int32),
            pltpu.VMEM((NBUF, K, d), jnp.float32),
            pltpu.VMEM_SHARED((NPAD, d), jnp.float32),
            pltpu.SemaphoreType.DMA((NBUF,)),
        ],
    )
    def agg(src_hbm, dst_hbm, zs_hbm, zeros_hbm, out_hbm, sidx, didx, sidx_x,
            didx_x, rows, acc, gsem):
        c = lax.axis_index("c")
        s = lax.axis_index("s")
        wid = c * ns + s
        pltpu.sync_copy(zeros_hbm.at[pl.ds(s * rpt, rpt)],
                        acc.at[pl.ds(s * rpt, rpt)])
        pltpu.sync_copy(src_hbm.at[pl.ds(wid * cw, cw)], sidx)
        pltpu.sync_copy(dst_hbm.at[pl.ds(wid * cw, cw)], didx)

        @pl.when(wid < nx)
        def _():
            pltpu.sync_copy(src_hbm.at[pl.ds(cw * nw + wid, 1)], sidx_x)
            pltpu.sync_copy(dst_hbm.at[pl.ds(cw * nw + wid, 1)], didx_x)

        plsc.subcore_barrier()

        for b in range(PD):
            pltpu.async_copy(zs_hbm.at[sidx.at[b]], rows.at[b], gsem.at[b])

        def body(j, carry):
            jb = j + PD
            b2 = lax.rem(jb, NBUF)

            @pl.when(jb < cw)
            def _():
                pltpu.async_copy(zs_hbm.at[sidx.at[jb]], rows.at[b2],
                                 gsem.at[b2])

            b = lax.rem(j, NBUF)
            pltpu.make_async_copy(zs_hbm.at[sidx.at[j]], rows.at[b],
                                  gsem.at[b]).wait()
            pltpu.sync_copy(rows.at[b], acc.at[didx.at[j]], add=True)
            return carry

        lax.fori_loop(0, cw, body, 0)

        @pl.when(wid < nx)
        def _():
            pltpu.sync_copy(zs_hbm.at[sidx_x.at[0]], rows.at[0])
            pltpu.sync_copy(rows.at[0], acc.at[didx_x.at[0]], add=True)

        plsc.subcore_barrier()
        pltpu.sync_copy(acc.at[pl.ds(s * rpt, rpt)],
                        out_hbm.at[c, pl.ds(s * rpt, rpt)])

    return agg


def _make_deg():
    info = plsc.get_sparse_core_info()
    nc, ns = info.num_cores, info.num_subcores
    nw = nc * ns
    cw = CHUNKS // nw
    nx = CHUNKS - cw * nw
    rpt = NPAD // ns

    mesh = plsc.VectorSubcoreMesh(core_axis_name="c", subcore_axis_name="s")

    @functools.partial(
        pl.kernel,
        mesh=mesh,
        out_type=jax.ShapeDtypeStruct((nc, NPAD, DEGW), jnp.float32),
        compiler_params=pltpu.CompilerParams(use_tc_tiling_on_sc=False),
        scratch_types=[
            pltpu.VMEM((cw, K), jnp.int32),
            pltpu.VMEM((1, K), jnp.int32),
            pltpu.VMEM((K, DEGW), jnp.float32),
            pltpu.VMEM_SHARED((NPAD, DEGW), jnp.float32),
        ],
    )
    def deg(dst_hbm, ones_hbm, zeros_hbm, out_hbm, didx, didx_x, ones, acc):
        c = lax.axis_index("c")
        s = lax.axis_index("s")
        wid = c * ns + s
        pltpu.sync_copy(zeros_hbm.at[pl.ds(s * rpt, rpt)],
                        acc.at[pl.ds(s * rpt, rpt)])
        pltpu.sync_copy(dst_hbm.at[pl.ds(wid * cw, cw)], didx)
        pltpu.sync_copy(ones_hbm, ones)

        @pl.when(wid < nx)
        def _():
            pltpu.sync_copy(dst_hbm.at[pl.ds(cw * nw + wid, 1)], didx_x)

        plsc.subcore_barrier()

        def body(j, carry):
            pltpu.sync_copy(ones, acc.at[didx.at[j]], add=True)
            return carry

        lax.fori_loop(0, cw, body, 0)

        @pl.when(wid < nx)
        def _():
            pltpu.sync_copy(ones, acc.at[didx_x.at[0]], add=True)

        plsc.subcore_barrier()
        pltpu.sync_copy(acc.at[pl.ds(s * rpt, rpt)],
                        out_hbm.at[c, pl.ds(s * rpt, rpt)])

    return deg


def _tc1a_body(x_ref, w_ref, z_ref):
    z_ref[...] = jnp.dot(x_ref[...], w_ref[...],
                         preferred_element_type=jnp.float32)


def _tc1b_body(degp_ref, z_ref, zs_ref, dinv_ref):
    deg = degp_ref[0] + degp_ref[1] + 1.0
    dinv = lax.rsqrt(deg)
    dinv_ref[...] = dinv
    zs_ref[0:N, :] = dinv[0:N] * z_ref[...]
    zs_ref[N:NPAD, :] = jnp.zeros((NPAD - N, 64), jnp.float32)


def _tc2_body(acc_ref, zs_ref, dinv_ref, b_ref, w_ref, out_ref):
    dinv = dinv_ref[...]
    agg = acc_ref[0] + acc_ref[1] + zs_ref[...]
    h = jnp.maximum(dinv * agg + b_ref[...], 0.0)
    z2 = jnp.dot(h[0:N], w_ref[...], preferred_element_type=jnp.float32)
    out_ref[0:N, :] = dinv[0:N] * z2
    out_ref[N:NPAD, :] = jnp.zeros((NPAD - N, 32), jnp.float32)


def _tc3_body(acc_ref, zs_ref, dinv_ref, b_ref, wc_ref, bc_ref, out_ref):
    dinv = dinv_ref[0:N]
    agg = acc_ref[0, 0:N] + acc_ref[1, 0:N] + zs_ref[0:N]
    h = jnp.maximum(dinv * agg + b_ref[...], 0.0)
    o = jnp.dot(h, wc_ref[...], preferred_element_type=jnp.float32) + bc_ref[...]
    out_ref[...] = jax.nn.sigmoid(o)


def kernel(x, edge_index, W1, b1, W2, b2, Wc, bc):
    src_p = edge_index[0].reshape(CHUNKS, K)
    dst_p = edge_index[1].reshape(CHUNKS, K)

    zeros64 = jnp.zeros((NPAD, 64), jnp.float32)
    zeros32 = jnp.zeros((NPAD, 32), jnp.float32)
    zerosd = jnp.zeros((NPAD, DEGW), jnp.float32)
    onesd = jnp.ones((K, DEGW), jnp.float32)

    degp = _make_deg()(dst_p, onesd, zerosd)[:, :, 0:1]

    tc1a = pl.pallas_call(
        _tc1a_body,
        out_shape=jax.ShapeDtypeStruct((N, 64), jnp.float32),
    )
    z1 = tc1a(x, W1)

    tc1b = pl.pallas_call(
        _tc1b_body,
        out_shape=(jax.ShapeDtypeStruct((NPAD, 64), jnp.float32),
                   jax.ShapeDtypeStruct((NPAD, 1), jnp.float32)),
    )
    zs1, dinv = tc1b(degp, z1)

    acc1 = _make_agg(64)(src_p, dst_p, zs1, zeros64)

    tc2 = pl.pallas_call(
        _tc2_body,
        out_shape=jax.ShapeDtypeStruct((NPAD, 32), jnp.float32),
    )
    zs2 = tc2(acc1, zs1, dinv, b1.reshape(1, 64), W2)

    acc2 = _make_agg(32)(src_p, dst_p, zs2, zeros32)

    tc3 = pl.pallas_call(
        _tc3_body,
        out_shape=jax.ShapeDtypeStruct((N, 1), jnp.float32),
    )
    return tc3(acc2, zs2, dinv, b2.reshape(1, 32), Wc, bc.reshape(1, 1))


# speedup vs baseline: 1.0254x; 1.0125x over previous
import functools

import jax
import jax.numpy as jnp
from jax import lax
from jax.experimental import pallas as pl
from jax.experimental.pallas import tpu as pltpu
from jax.experimental.pallas import tpu_sc as plsc

N = 10000
E = 320000
NPAD = 10112
K = 128
CHUNKS = E // K
DEGW = 8
NBUF = 8
PD = 4


def _make_agg(d):
    info = plsc.get_sparse_core_info()
    nc, ns = info.num_cores, info.num_subcores
    nw = nc * ns
    cw = CHUNKS // nw
    nx = CHUNKS - cw * nw
    rpt = NPAD // ns

    mesh = plsc.VectorSubcoreMesh(core_axis_name="c", subcore_axis_name="s")

    @functools.partial(
        pl.kernel,
        mesh=mesh,
        out_type=jax.ShapeDtypeStruct((nc, NPAD, d), jnp.float32),
        compiler_params=pltpu.CompilerParams(use_tc_tiling_on_sc=False),
        scratch_types=[
            pltpu.VMEM((cw, K), jnp.int32),
            pltpu.VMEM((cw, K), jnp.int32),
            pltpu.VMEM((1, K), jnp.int32),
            pltpu.VMEM((1, K), jnp.int32),
            pltpu.VMEM((NBUF, K, d), jnp.float32),
            pltpu.VMEM_SHARED((NPAD, d), jnp.float32),
            pltpu.SemaphoreType.DMA((NBUF,)),
        ],
    )
    def agg(src_hbm, dst_hbm, zs_hbm, zeros_hbm, out_hbm, sidx, didx, sidx_x,
            didx_x, rows, acc, gsem):
        c = lax.axis_index("c")
        s = lax.axis_index("s")
        wid = c * ns + s
        pltpu.sync_copy(zeros_hbm.at[pl.ds(s * rpt, rpt)],
                        acc.at[pl.ds(s * rpt, rpt)])
        pltpu.sync_copy(src_hbm.at[pl.ds(wid * cw, cw)], sidx)
        pltpu.sync_copy(dst_hbm.at[pl.ds(wid * cw, cw)], didx)

        @pl.when(wid < nx)
        def _():
            pltpu.sync_copy(src_hbm.at[pl.ds(cw * nw + wid, 1)], sidx_x)
            pltpu.sync_copy(dst_hbm.at[pl.ds(cw * nw + wid, 1)], didx_x)

        plsc.subcore_barrier()

        for b in range(PD):
            pltpu.async_copy(zs_hbm.at[sidx.at[b]], rows.at[b], gsem.at[b])

        def body(j, carry):
            jb = j + PD
            b2 = lax.rem(jb, NBUF)

            @pl.when(jb < cw)
            def _():
                pltpu.async_copy(zs_hbm.at[sidx.at[jb]], rows.at[b2],
                                 gsem.at[b2])

            b = lax.rem(j, NBUF)
            pltpu.make_async_copy(zs_hbm.at[sidx.at[j]], rows.at[b],
                                  gsem.at[b]).wait()
            pltpu.sync_copy(rows.at[b], acc.at[didx.at[j]], add=True)
            return carry

        lax.fori_loop(0, cw, body, 0)

        @pl.when(wid < nx)
        def _():
            pltpu.sync_copy(zs_hbm.at[sidx_x.at[0]], rows.at[0])
            pltpu.sync_copy(rows.at[0], acc.at[didx_x.at[0]], add=True)

        plsc.subcore_barrier()
        pltpu.sync_copy(acc.at[pl.ds(s * rpt, rpt)],
                        out_hbm.at[c, pl.ds(s * rpt, rpt)])

    return agg


def _make_deg():
    info = plsc.get_sparse_core_info()
    nc, ns = info.num_cores, info.num_subcores
    nw = nc * ns
    cw = CHUNKS // nw
    nx = CHUNKS - cw * nw
    rpt = NPAD // ns

    mesh = plsc.VectorSubcoreMesh(core_axis_name="c", subcore_axis_name="s")

    @functools.partial(
        pl.kernel,
        mesh=mesh,
        out_type=jax.ShapeDtypeStruct((nc, NPAD, DEGW), jnp.float32),
        compiler_params=pltpu.CompilerParams(use_tc_tiling_on_sc=False),
        scratch_types=[
            pltpu.VMEM((cw, K), jnp.int32),
            pltpu.VMEM((1, K), jnp.int32),
            pltpu.VMEM((K, DEGW), jnp.float32),
            pltpu.VMEM_SHARED((NPAD, DEGW), jnp.float32),
        ],
    )
    def deg(dst_hbm, ones_hbm, zeros_hbm, out_hbm, didx, didx_x, ones, acc):
        c = lax.axis_index("c")
        s = lax.axis_index("s")
        wid = c * ns + s
        pltpu.sync_copy(zeros_hbm.at[pl.ds(s * rpt, rpt)],
                        acc.at[pl.ds(s * rpt, rpt)])
        pltpu.sync_copy(dst_hbm.at[pl.ds(wid * cw, cw)], didx)
        pltpu.sync_copy(ones_hbm, ones)

        @pl.when(wid < nx)
        def _():
            pltpu.sync_copy(dst_hbm.at[pl.ds(cw * nw + wid, 1)], didx_x)

        plsc.subcore_barrier()

        def body(j, carry):
            pltpu.sync_copy(ones, acc.at[didx.at[j]], add=True)
            return carry

        lax.fori_loop(0, cw, body, 0)

        @pl.when(wid < nx)
        def _():
            pltpu.sync_copy(ones, acc.at[didx_x.at[0]], add=True)

        plsc.subcore_barrier()
        pltpu.sync_copy(acc.at[pl.ds(s * rpt, rpt)],
                        out_hbm.at[c, pl.ds(s * rpt, rpt)])

    return deg


def _tc1a_body(x_ref, w_ref, z_ref):
    z_ref[...] = jnp.dot(x_ref[...], w_ref[...],
                         preferred_element_type=jnp.float32)


def _tc1b_body(degp_ref, z_ref, zs_ref):
    deg = degp_ref[0] + degp_ref[1] + 1.0
    dinv = lax.rsqrt(deg)
    zs_ref[:, 64:65] = dinv
    zs_ref[0:N, 0:64] = dinv[0:N] * z_ref[...]
    zs_ref[N:NPAD, 0:64] = jnp.zeros((NPAD - N, 64), jnp.float32)


def _tc2_body(acc_ref, zs_ref, b_ref, w_ref, out_ref):
    dinv = zs_ref[:, 64:65]
    agg = acc_ref[0] + acc_ref[1] + zs_ref[:, 0:64]
    h = jnp.maximum(dinv * agg + b_ref[...], 0.0)
    z2 = jnp.dot(h[0:N], w_ref[...], preferred_element_type=jnp.float32)
    out_ref[:, 32:33] = dinv
    out_ref[0:N, 0:32] = dinv[0:N] * z2
    out_ref[N:NPAD, 0:32] = jnp.zeros((NPAD - N, 32), jnp.float32)


def _tc3_body(acc_ref, zs_ref, b_ref, wc_ref, bc_ref, out_ref):
    dinv = zs_ref[0:N, 32:33]
    agg = acc_ref[0, 0:N] + acc_ref[1, 0:N] + zs_ref[0:N, 0:32]
    h = jnp.maximum(dinv * agg + b_ref[...], 0.0)
    o = jnp.dot(h, wc_ref[...], preferred_element_type=jnp.float32) + bc_ref[...]
    out_ref[...] = jax.nn.sigmoid(o)


def kernel(x, edge_index, W1, b1, W2, b2, Wc, bc):
    src_p = edge_index[0].reshape(CHUNKS, K)
    dst_p = edge_index[1].reshape(CHUNKS, K)

    zeros64 = jnp.zeros((NPAD, 64), jnp.float32)
    zeros32 = jnp.zeros((NPAD, 32), jnp.float32)
    zerosd = jnp.zeros((NPAD, DEGW), jnp.float32)
    onesd = jnp.ones((K, DEGW), jnp.float32)

    degp = _make_deg()(dst_p, onesd, zerosd)[:, :, 0:1]

    tc1a = pl.pallas_call(
        _tc1a_body,
        out_shape=jax.ShapeDtypeStruct((N, 64), jnp.float32),
    )
    z1 = tc1a(x, W1)

    tc1b = pl.pallas_call(
        _tc1b_body,
        out_shape=jax.ShapeDtypeStruct((NPAD, 65), jnp.float32),
    )
    zsx1 = tc1b(degp, z1)

    acc1 = _make_agg(64)(src_p, dst_p, zsx1[:, 0:64], zeros64)

    tc2 = pl.pallas_call(
        _tc2_body,
        out_shape=jax.ShapeDtypeStruct((NPAD, 33), jnp.float32),
    )
    zsx2 = tc2(acc1, zsx1, b1.reshape(1, 64), W2)

    acc2 = _make_agg(32)(src_p, dst_p, zsx2[:, 0:32], zeros32)

    tc3 = pl.pallas_call(
        _tc3_body,
        out_shape=jax.ShapeDtypeStruct((N, 1), jnp.float32),
    )
    return tc3(acc2, zsx2, b2.reshape(1, 32), Wc, bc.reshape(1, 1))
